# Initial kernel scaffold; baseline (speedup 1.0000x reference)
import jax
import jax.numpy as jnp
from jax.experimental import pallas as pl


def kernel(x, edge_index, edge_attr, W_in, b_in, W1, b1, g1, be1, W2, b2, g2, be2, W3, b3, g3, be3, Wf, bf, gf, bef, gates):
    raise NotImplementedError("write your pallas kernel here")


# speedup vs baseline: 1.0482x; 1.0482x over previous
import jax
import jax.numpy as jnp
from jax.experimental import pallas as pl

N = 10000
H = 128


def _ln(v, g, b):
    mu = jnp.mean(v, axis=-1, keepdims=True)
    var = jnp.mean((v - mu) ** 2, axis=-1, keepdims=True)
    return (v - mu) / jnp.sqrt(var + 1e-5) * g + b


def _in_body(x_ref, w_ref, b_ref, o_ref):
    o_ref[...] = jax.nn.relu(
        jnp.dot(x_ref[...], w_ref[...], preferred_element_type=jnp.float32)
        + b_ref[...]
    )


def _layer_body(agg_ref, h_ref, w_ref, b_ref, g_ref, be_ref, o_ref):
    agg = agg_ref[...]
    agg = jnp.where(jnp.isneginf(agg), 0.0, agg)
    out = jnp.dot(agg, w_ref[...], preferred_element_type=jnp.float32) + b_ref[...]
    out = _ln(out, g_ref[...], be_ref[...])
    o_ref[...] = jax.nn.relu(out + h_ref[...])


def _final_body(aggf_ref, h1_ref, h2_ref, h3_ref, x0_ref, gates_ref,
                w_ref, b_ref, g_ref, be_ref, o_ref):
    aggf = aggf_ref[...]
    sinkf = jnp.logical_not(jnp.isneginf(aggf[:, 0:1]))
    aggf = jnp.where(jnp.isneginf(aggf), 0.0, aggf)
    s = jax.nn.sigmoid(gates_ref[...])
    nf = h1_ref[...] * s[0:1, :] + h2_ref[...] * s[1:2, :] + h3_ref[...] * s[2:3, :]
    out = jnp.dot(aggf, w_ref[...], preferred_element_type=jnp.float32) + b_ref[...]
    out = _ln(out, g_ref[...], be_ref[...])
    subf = jax.nn.relu(out + nf)
    neg = jnp.float32(-jnp.inf)
    v2m = jnp.max(jnp.where(sinkf, subf, neg), axis=0, keepdims=True)
    any_sink = jnp.any(sinkf)
    v2 = jnp.where(any_sink, v2m, jnp.full((1, H), 1e-4, jnp.float32))
    rem = jnp.logical_not(x0_ref[...] > 0.1)
    v3m = jnp.max(jnp.where(rem, nf, neg), axis=0, keepdims=True)
    any_rem = jnp.any(rem)
    v3 = jnp.where(any_rem, v3m, jnp.full((1, H), 1e-4, jnp.float32))
    o_ref[0:1, :] = v2
    o_ref[1:2, :] = v3


def _tc_in(x, W, b):
    return pl.pallas_call(
        _in_body,
        out_shape=jax.ShapeDtypeStruct((N, H), jnp.float32),
    )(x, W, b.reshape(1, H))


def _tc_layer(agg, h, W, b, g, be):
    return pl.pallas_call(
        _layer_body,
        out_shape=jax.ShapeDtypeStruct((N, H), jnp.float32),
    )(agg, h, W, b.reshape(1, H), g.reshape(1, H), be.reshape(1, H))


def _tc_final(aggf, h1, h2, h3, x0, gates, W, b, g, be):
    return pl.pallas_call(
        _final_body,
        out_shape=jax.ShapeDtypeStruct((2, H), jnp.float32),
    )(aggf, h1, h2, h3, x0, gates, W, b.reshape(1, H), g.reshape(1, H),
      be.reshape(1, H))


def _segmax(vals_src, src, dst, mask=None):
    rows = vals_src[src]
    if mask is not None:
        rows = jnp.where(mask[:, None], rows, -jnp.inf)
    return jax.ops.segment_max(rows, dst, num_segments=N)


def kernel(x, edge_index, edge_attr, W_in, b_in, W1, b1, g1, be1, W2, b2, g2,
           be2, W3, b3, g3, be3, Wf, bf, gf, bef, gates):
    src = edge_index[0]
    dst = edge_index[1]
    h0 = _tc_in(x, W_in, b_in)
    agg1 = _segmax(h0, src, dst)
    h1 = _tc_layer(agg1, h0, W1, b1, g1, be1)
    agg2 = _segmax(h1, src, dst)
    h2 = _tc_layer(agg2, h1, W2, b2, g2, be2)
    agg3 = _segmax(h2, src, dst)
    h3 = _tc_layer(agg3, h2, W3, b3, g3, be3)
    s = jax.nn.sigmoid(gates)
    nf = h1 * s[0] + h2 * s[1] + h3 * s[2]
    aggf = _segmax(nf, src, dst, mask=(edge_attr == 3))
    return _tc_final(aggf, h1, h2, h3, x[:, 0:1], gates, Wf, bf, gf, bef)


# speedup vs baseline: 1.2850x; 1.2259x over previous
import functools

import jax
import jax.numpy as jnp
from jax import lax
from jax.experimental import pallas as pl
from jax.experimental.pallas import tpu as pltpu
from jax.experimental.pallas import tpu_sc as plsc

N = 10000
E = 320000
H = 128


NC = 2
NS = 16
LANES = 16
NWORK = NC * NS
OWN = 313
NPT = OWN + 1
CH = 4000
NCHUNK = E // CH
RING = 4


def _segmax_sc_body(masked, h_hbm, src_hbm, dst_hbm, ea_hbm, agg_hbm,
                    srcv, dstv, eav, csrc, cldst, acc, rows,
                    sem0, sem1, sem2, sem3):
    sems = (sem0, sem1, sem2, sem3)
    wid = lax.axis_index("s") * NC + lax.axis_index("c")
    lo = wid * OWN
    hi = jnp.minimum(lo + OWN, N)
    neg = jnp.full((LANES,), -jnp.inf, jnp.float32)
    iota = lax.iota(jnp.int32, LANES)

    def initb(i, c):
        acc[pl.ds(i * LANES, LANES)] = neg
        return c
    lax.fori_loop(0, (NPT * H) // LANES, initb, jnp.int32(0))

    def issue(w, r):
        srcw = csrc[pl.ds(w * LANES, LANES)]
        pltpu.async_copy(h_hbm.at[srcw], rows.at[r], sems[r])

    def chunk_body(ci, carry):
        base = ci * CH
        pltpu.sync_copy(src_hbm.at[pl.ds(base, CH)], srcv)
        pltpu.sync_copy(dst_hbm.at[pl.ds(base, CH)], dstv)
        if masked:
            pltpu.sync_copy(ea_hbm.at[pl.ds(base, CH)], eav)

        def scanb(i, cnt):
            dv = dstv[pl.ds(i * LANES, LANES)]
            sv = srcv[pl.ds(i * LANES, LANES)]
            m = (dv >= lo) & (dv < hi)
            if masked:
                m = m & (eav[pl.ds(i * LANES, LANES)] == 3)
            mi = m.astype(jnp.int32)
            pos = cnt + jnp.cumsum(mi) - 1
            plsc.store_scatter(csrc, [pos], sv, mask=m)
            plsc.store_scatter(cldst, [pos], dv - lo, mask=m)
            return cnt + jnp.sum(mi)
        cnt = lax.fori_loop(0, CH // LANES, scanb, jnp.int32(0))

        csrc[pl.ds(cnt, LANES)] = jnp.zeros((LANES,), jnp.int32)
        cldst[pl.ds(cnt, LANES)] = jnp.full((LANES,), OWN, jnp.int32)
        nwave = (cnt + LANES - 1) // LANES

        for r in range(RING):
            @pl.when(r < nwave)
            def _():
                issue(jnp.int32(r), r)

        def wave_outer(wo, carry2):
            for r in range(RING):
                w = wo * RING + r

                @pl.when(w < nwave)
                def _():
                    srcw = csrc[pl.ds(w * LANES, LANES)]
                    pltpu.make_async_copy(h_hbm.at[srcw], rows.at[r],
                                          sems[r]).wait()
                    rr = rows.at[r]
                    for e in range(LANES):
                        dbc = plsc.load_gather(
                            cldst,
                            [jnp.broadcast_to(w * LANES + e, (LANES,))])
                        bidx = dbc * H + iota
                        for c in range(H // LANES):
                            idx = bidx + c * LANES
                            cur = plsc.load_gather(acc, [idx])
                            val = rr[e, pl.ds(c * LANES, LANES)]
                            plsc.store_scatter(acc, [idx],
                                               jnp.maximum(cur, val))

                    @pl.when(w + RING < nwave)
                    def _():
                        issue(w + RING, r)
            return carry2
        lax.fori_loop(0, (nwave + RING - 1) // RING, wave_outer,
                      jnp.int32(0))
        return carry
    lax.fori_loop(0, NCHUNK, chunk_body, jnp.int32(0))

    pltpu.sync_copy(acc.at[pl.ds(0, OWN * H)],
                    agg_hbm.at[pl.ds(lo * H, OWN * H)])


def _make_segmax(masked):
    mesh = plsc.VectorSubcoreMesh(core_axis_name="c", subcore_axis_name="s",
                                  num_cores=NC, num_subcores=NS)
    return pl.kernel(
        functools.partial(_segmax_sc_body, masked),
        out_type=jax.ShapeDtypeStruct((NWORK * OWN * H,), jnp.float32),
        mesh=mesh,
        compiler_params=pltpu.CompilerParams(needs_layout_passes=False),
        scratch_types=[
            pltpu.VMEM((CH,), jnp.int32),
            pltpu.VMEM((CH,), jnp.int32),
            pltpu.VMEM((CH,), jnp.int32),
            pltpu.VMEM((CH + LANES,), jnp.int32),
            pltpu.VMEM((CH + LANES,), jnp.int32),
            pltpu.VMEM((NPT * H,), jnp.float32),
            pltpu.VMEM((RING, LANES, H), jnp.float32),
            pltpu.SemaphoreType.DMA,
            pltpu.SemaphoreType.DMA,
            pltpu.SemaphoreType.DMA,
            pltpu.SemaphoreType.DMA,
        ],
    )


_segmax_plain = _make_segmax(False)
_segmax_masked = _make_segmax(True)


def _sc_segmax(h, src, dst, ea, masked):
    fn = _segmax_masked if masked else _segmax_plain
    flat = fn(h, src, dst, ea)
    return flat[:N * H].reshape(N, H)


def _ln(v, g, b):
    mu = jnp.mean(v, axis=-1, keepdims=True)
    var = jnp.mean((v - mu) ** 2, axis=-1, keepdims=True)
    return (v - mu) / jnp.sqrt(var + 1e-5) * g + b


def _in_body(x_ref, w_ref, b_ref, o_ref):
    o_ref[...] = jax.nn.relu(
        jnp.dot(x_ref[...], w_ref[...], preferred_element_type=jnp.float32)
        + b_ref[...]
    )


def _layer_body(agg_ref, h_ref, w_ref, b_ref, g_ref, be_ref, o_ref):
    agg = agg_ref[...]
    agg = jnp.where(jnp.isneginf(agg), 0.0, agg)
    out = jnp.dot(agg, w_ref[...], preferred_element_type=jnp.float32) + b_ref[...]
    out = _ln(out, g_ref[...], be_ref[...])
    o_ref[...] = jax.nn.relu(out + h_ref[...])


def _layer3_body(agg_ref, h1_ref, h2_ref, w_ref, b_ref, g_ref, be_ref,
                 gates_ref, nf_ref):
    agg = agg_ref[...]
    agg = jnp.where(jnp.isneginf(agg), 0.0, agg)
    out = jnp.dot(agg, w_ref[...], preferred_element_type=jnp.float32) + b_ref[...]
    out = _ln(out, g_ref[...], be_ref[...])
    h3 = jax.nn.relu(out + h2_ref[...])
    s = jax.nn.sigmoid(gates_ref[...])
    nf_ref[...] = (h1_ref[...] * s[0:1, :] + h2_ref[...] * s[1:2, :]
                   + h3 * s[2:3, :])


def _final_body(aggf_ref, nf_ref, x0_ref, w_ref, b_ref, g_ref, be_ref, o_ref):
    aggf = aggf_ref[...]
    sinkf = jnp.logical_not(jnp.isneginf(aggf[:, 0:1]))
    aggf = jnp.where(jnp.isneginf(aggf), 0.0, aggf)
    nf = nf_ref[...]
    out = jnp.dot(aggf, w_ref[...], preferred_element_type=jnp.float32) + b_ref[...]
    out = _ln(out, g_ref[...], be_ref[...])
    subf = jax.nn.relu(out + nf)
    neginf = jnp.float32(-jnp.inf)
    v2m = jnp.max(jnp.where(sinkf, subf, neginf), axis=0, keepdims=True)
    v2 = jnp.where(jnp.any(sinkf), v2m, jnp.full((1, H), 1e-4, jnp.float32))
    rem = jnp.logical_not(x0_ref[...] > 0.1)
    v3m = jnp.max(jnp.where(rem, nf, neginf), axis=0, keepdims=True)
    v3 = jnp.where(jnp.any(rem), v3m, jnp.full((1, H), 1e-4, jnp.float32))
    o_ref[0:1, :] = v2
    o_ref[1:2, :] = v3


def _tc_in(x, W, b):
    return pl.pallas_call(
        _in_body,
        out_shape=jax.ShapeDtypeStruct((N, H), jnp.float32),
    )(x, W, b.reshape(1, H))


def _tc_layer(agg, h, W, b, g, be):
    return pl.pallas_call(
        _layer_body,
        out_shape=jax.ShapeDtypeStruct((N, H), jnp.float32),
    )(agg, h, W, b.reshape(1, H), g.reshape(1, H), be.reshape(1, H))


def _tc_layer3(agg, h1, h2, W, b, g, be, gates):
    return pl.pallas_call(
        _layer3_body,
        out_shape=jax.ShapeDtypeStruct((N, H), jnp.float32),
    )(agg, h1, h2, W, b.reshape(1, H), g.reshape(1, H), be.reshape(1, H),
      gates)


def _tc_final(aggf, nf, x0, W, b, g, be):
    return pl.pallas_call(
        _final_body,
        out_shape=jax.ShapeDtypeStruct((2, H), jnp.float32),
    )(aggf, nf, x0, W, b.reshape(1, H), g.reshape(1, H), be.reshape(1, H))


def kernel(x, edge_index, edge_attr, W_in, b_in, W1, b1, g1, be1, W2, b2, g2,
           be2, W3, b3, g3, be3, Wf, bf, gf, bef, gates):
    src = edge_index[0]
    dst = edge_index[1]
    h0 = _tc_in(x, W_in, b_in)
    agg1 = _sc_segmax(h0, src, dst, edge_attr, False)
    h1 = _tc_layer(agg1, h0, W1, b1, g1, be1)
    agg2 = _sc_segmax(h1, src, dst, edge_attr, False)
    h2 = _tc_layer(agg2, h1, W2, b2, g2, be2)
    agg3 = _sc_segmax(h2, src, dst, edge_attr, False)
    nf = _tc_layer3(agg3, h1, h2, W3, b3, g3, be3, gates)
    aggf = _sc_segmax(nf, src, dst, edge_attr, True)
    return _tc_final(aggf, nf, x[:, 0:1], Wf, bf, gf, bef)


# speedup vs baseline: 2.5546x; 1.9880x over previous
import functools

import jax
import jax.numpy as jnp
from jax import lax
from jax.experimental import pallas as pl
from jax.experimental.pallas import tpu as pltpu
from jax.experimental.pallas import tpu_sc as plsc

N = 10000
E = 320000
H = 128

NC = 2
NS = 16
LANES = 16
NWORK = NC * NS
OWN = 313
NPT = OWN + 1
SHIFT = 14
SMASK = (1 << SHIFT) - 1

CP = 8000
NCHUNKP = E // CP
REG = E + 12288

B = 4096
WAVE = 64
RING = 4


def _sc_mesh():
    return plsc.VectorSubcoreMesh(core_axis_name="c", subcore_axis_name="s",
                                  num_cores=NC, num_subcores=NS)


def _wid():
    return lax.axis_index("s") * NC + lax.axis_index("c")


def _prep_body(src_hbm, dst_hbm, ea_hbm,
               lst_hbm, plst_hbm, cnt_hbm, pcnt_hbm,
               srcv0, srcv1, dstv0, dstv1, eav0, eav1,
               cpk0, cpk1, cpm0, cpm1, cstg,
               l0, l1, l2, l3, l4, l5, f0, f1, f2, f3):
    lsems = ((l0, l1, l2), (l3, l4, l5))
    fsems = ((f0, f1), (f2, f3))
    srcvs = (srcv0, srcv1)
    dstvs = (dstv0, dstv1)
    eavs = (eav0, eav1)
    cpks = (cpk0, cpk1)
    cpms = (cpm0, cpm1)
    wid = _wid()
    lo = wid * OWN
    hi = jnp.minimum(lo + OWN, N)
    tbase = wid * REG
    iota = lax.iota(jnp.int32, LANES)
    dummy = jnp.full((LANES,), OWN << SHIFT, jnp.int32)
    trash = jnp.int32(CP + 16) + iota

    def issue_loads(ci, b):
        pltpu.async_copy(src_hbm.at[pl.ds(ci * CP, CP)], srcvs[b],
                         lsems[b][0])
        pltpu.async_copy(dst_hbm.at[pl.ds(ci * CP, CP)], dstvs[b],
                         lsems[b][1])
        pltpu.async_copy(ea_hbm.at[pl.ds(ci * CP, CP)], eavs[b],
                         lsems[b][2])

    def wait_loads(ci, b):
        pltpu.make_async_copy(src_hbm.at[pl.ds(ci * CP, CP)], srcvs[b],
                              lsems[b][0]).wait()
        pltpu.make_async_copy(dst_hbm.at[pl.ds(ci * CP, CP)], dstvs[b],
                              lsems[b][1]).wait()
        pltpu.make_async_copy(ea_hbm.at[pl.ds(ci * CP, CP)], eavs[b],
                              lsems[b][2]).wait()

    def wait_flush(b):
        pltpu.make_async_copy(cpks[b], lst_hbm.at[pl.ds(tbase, CP + 32)],
                              fsems[b][0]).wait()
        pltpu.make_async_copy(cpms[b], plst_hbm.at[pl.ds(tbase, CP + 32)],
                              fsems[b][1]).wait()

    issue_loads(jnp.int32(0), 0)
    issue_loads(jnp.int32(1), 1)

    def pair_body(cp_i, offs):
        off, offp = offs
        for b in (0, 1):
            ci = cp_i * 2 + b
            wait_loads(ci, b)

            @pl.when(ci >= 2)
            def _():
                wait_flush(b)

            def scan(g, cnts):
                cnt, cntp = cnts
                sv = srcvs[b][pl.ds(g * LANES, LANES)]
                dv = dstvs[b][pl.ds(g * LANES, LANES)]
                ev = eavs[b][pl.ds(g * LANES, LANES)]
                m = (dv >= lo) & (dv < hi)
                pk = sv | ((dv - lo) << SHIFT)
                mi = jnp.where(m, jnp.ones((LANES,), jnp.int32),
                               jnp.zeros((LANES,), jnp.int32))
                cs = jnp.cumsum(mi)
                pos = jnp.where(m, cnt + cs - 1, trash)
                plsc.store_scatter(cpks[b], [pos], pk)
                mp = m & (ev == 3)
                mpi = jnp.where(mp, jnp.ones((LANES,), jnp.int32),
                                jnp.zeros((LANES,), jnp.int32))
                cs2 = jnp.cumsum(mpi)
                pos2 = jnp.where(mp, cntp + cs2 - 1, trash)
                plsc.store_scatter(cpms[b], [pos2], pk)
                return (cnt + cs[LANES - 1], cntp + cs2[LANES - 1])

            cnt, cntp = lax.fori_loop(0, CP // LANES, scan,
                                      (jnp.int32(0), jnp.int32(0)))
            plsc.store_scatter(cpks[b], [cnt + iota], dummy)
            plsc.store_scatter(cpms[b], [cntp + iota], dummy)
            fo = pl.multiple_of(tbase + off, 16)
            fop = pl.multiple_of(tbase + offp, 16)
            pltpu.async_copy(cpks[b], lst_hbm.at[pl.ds(fo, CP + 32)],
                             fsems[b][0])
            pltpu.async_copy(cpms[b], plst_hbm.at[pl.ds(fop, CP + 32)],
                             fsems[b][1])
            off = off + ((cnt + 15) // 16) * 16
            offp = offp + ((cntp + 15) // 16) * 16

            @pl.when(ci + 2 < NCHUNKP)
            def _():
                issue_loads(ci + 2, b)
        return (off, offp)

    off, offp = lax.fori_loop(0, NCHUNKP // 2, pair_body,
                              (jnp.int32(0), jnp.int32(0)))
    for b in (0, 1):
        wait_flush(b)
    for g in range(4):
        cpk0[pl.ds(g * LANES, LANES)] = dummy
    pltpu.sync_copy(cpk0.at[pl.ds(0, 64)],
                    lst_hbm.at[pl.ds(pl.multiple_of(tbase + off, 16), 64)])
    pltpu.sync_copy(cpk0.at[pl.ds(0, 64)],
                    plst_hbm.at[pl.ds(pl.multiple_of(tbase + offp, 16), 64)])
    off_out = ((off + 63) // 64) * 64
    offp_out = ((offp + 63) // 64) * 64
    zeros = jnp.zeros((LANES,), jnp.int32)
    cstg[pl.ds(0, LANES)] = jnp.where(iota == 0, off_out, zeros)
    pltpu.sync_copy(cstg, cnt_hbm.at[pl.ds(wid * LANES, LANES)])
    cstg[pl.ds(0, LANES)] = jnp.where(iota == 0, offp_out, zeros)
    pltpu.sync_copy(cstg, pcnt_hbm.at[pl.ds(wid * LANES, LANES)])


_sc_prepare = pl.kernel(
    _prep_body,
    out_type=(
        jax.ShapeDtypeStruct((NWORK * REG,), jnp.int32),
        jax.ShapeDtypeStruct((NWORK * REG,), jnp.int32),
        jax.ShapeDtypeStruct((NWORK * LANES,), jnp.int32),
        jax.ShapeDtypeStruct((NWORK * LANES,), jnp.int32),
    ),
    mesh=_sc_mesh(),
    compiler_params=pltpu.CompilerParams(needs_layout_passes=False),
    scratch_types=[
        pltpu.VMEM((CP,), jnp.int32),
        pltpu.VMEM((CP,), jnp.int32),
        pltpu.VMEM((CP,), jnp.int32),
        pltpu.VMEM((CP,), jnp.int32),
        pltpu.VMEM((CP,), jnp.int32),
        pltpu.VMEM((CP,), jnp.int32),
        pltpu.VMEM((CP + 32,), jnp.int32),
        pltpu.VMEM((CP + 32,), jnp.int32),
        pltpu.VMEM((CP + 32,), jnp.int32),
        pltpu.VMEM((CP + 32,), jnp.int32),
        pltpu.VMEM((LANES,), jnp.int32),
    ] + [pltpu.SemaphoreType.DMA] * 10,
)


def _apply_body(h_hbm, lst_hbm, cnt_hbm, agg_hbm,
                lstv, bsrc, acc, rows, cstg, s0, s1, s2, s3):
    sems = (s0, s1, s2, s3)
    wid = _wid()
    lo = wid * OWN
    tbase = wid * REG
    iota = lax.iota(jnp.int32, LANES)
    neg = jnp.full((LANES,), -jnp.inf, jnp.float32)

    def initb(i, c):
        acc[pl.ds(i * LANES, LANES)] = neg
        return c
    lax.fori_loop(0, (NPT * H) // LANES, initb, jnp.int32(0))

    pltpu.sync_copy(cnt_hbm.at[pl.ds(wid * LANES, LANES)], cstg)
    off = cstg[pl.ds(0, LANES)][0]
    nblk = (off + B - 1) // B

    def issue(w, r):
        pltpu.async_copy(h_hbm.at[bsrc.at[pl.ds(w * WAVE, WAVE)]],
                         rows.at[r], sems[r])

    def wait_rows(w, r):
        pltpu.make_async_copy(h_hbm.at[bsrc.at[pl.ds(w * WAVE, WAVE)]],
                              rows.at[r], sems[r]).wait()

    def blk_body(bi, c):
        boff = bi * B
        pltpu.sync_copy(
            lst_hbm.at[pl.ds(pl.multiple_of(tbase + boff, 16), B)], lstv)

        def unpack(g, c2):
            bsrc[pl.ds(g * LANES, LANES)] = (lstv[pl.ds(g * LANES, LANES)]
                                             & SMASK)
            return c2
        lax.fori_loop(0, B // LANES, unpack, jnp.int32(0))
        nw = jnp.minimum(off - boff, B) // WAVE

        for r in range(RING):
            @pl.when(r < nw)
            def _():
                issue(jnp.int32(r), r)

        def wave_outer(wo, c3):
            for r in range(RING):
                w = wo * RING + r

                @pl.when(w < nw)
                def _():
                    wait_rows(w, r)

                    def edge(e, c4):
                        pk = plsc.load_gather(
                            lstv,
                            [jnp.broadcast_to(w * WAVE + e, (LANES,))])
                        dbc = lax.shift_right_logical(pk, SHIFT)
                        bidx = dbc * H + iota
                        esp = jnp.broadcast_to(e, (LANES,))
                        for cc in range(H // LANES):
                            idx = bidx + cc * LANES
                            cur = plsc.load_gather(acc, [idx])
                            val = plsc.load_gather(
                                rows.at[r], [esp, iota + cc * LANES])
                            plsc.store_scatter(acc, [idx],
                                               jnp.maximum(cur, val))
                        return c4
                    lax.fori_loop(0, WAVE, edge, jnp.int32(0))

                    @pl.when(w + RING < nw)
                    def _():
                        issue(w + RING, r)
            return c3
        lax.fori_loop(0, (nw + RING - 1) // RING, wave_outer, jnp.int32(0))
        return c
    lax.fori_loop(0, nblk, blk_body, jnp.int32(0))

    pltpu.sync_copy(acc.at[pl.ds(0, OWN * H)],
                    agg_hbm.at[pl.ds(lo * H, OWN * H)])


_sc_apply_k = pl.kernel(
    _apply_body,
    out_type=jax.ShapeDtypeStruct((NWORK * OWN * H,), jnp.float32),
    mesh=_sc_mesh(),
    compiler_params=pltpu.CompilerParams(needs_layout_passes=False),
    scratch_types=[
        pltpu.VMEM((B,), jnp.int32),
        pltpu.VMEM((B,), jnp.int32),
        pltpu.VMEM((NPT * H,), jnp.float32),
        pltpu.VMEM((RING, WAVE, H), jnp.float32),
        pltpu.VMEM((LANES,), jnp.int32),
    ] + [pltpu.SemaphoreType.DMA] * RING,
)


def _sc_apply(h, lst, cnt):
    flat = _sc_apply_k(h, lst, cnt)
    return flat[:N * H].reshape(N, H)


def _ln(v, g, b):
    mu = jnp.mean(v, axis=-1, keepdims=True)
    var = jnp.mean((v - mu) ** 2, axis=-1, keepdims=True)
    return (v - mu) / jnp.sqrt(var + 1e-5) * g + b


def _in_body(x_ref, w_ref, b_ref, o_ref):
    o_ref[...] = jax.nn.relu(
        jnp.dot(x_ref[...], w_ref[...], preferred_element_type=jnp.float32)
        + b_ref[...]
    )


def _layer_body(agg_ref, h_ref, w_ref, b_ref, g_ref, be_ref, o_ref):
    agg = agg_ref[...]
    agg = jnp.where(jnp.isneginf(agg), 0.0, agg)
    out = jnp.dot(agg, w_ref[...], preferred_element_type=jnp.float32) + b_ref[...]
    out = _ln(out, g_ref[...], be_ref[...])
    o_ref[...] = jax.nn.relu(out + h_ref[...])


def _layer3_body(agg_ref, h1_ref, h2_ref, w_ref, b_ref, g_ref, be_ref,
                 gates_ref, nf_ref):
    agg = agg_ref[...]
    agg = jnp.where(jnp.isneginf(agg), 0.0, agg)
    out = jnp.dot(agg, w_ref[...], preferred_element_type=jnp.float32) + b_ref[...]
    out = _ln(out, g_ref[...], be_ref[...])
    h3 = jax.nn.relu(out + h2_ref[...])
    s = jax.nn.sigmoid(gates_ref[...])
    nf_ref[...] = (h1_ref[...] * s[0:1, :] + h2_ref[...] * s[1:2, :]
                   + h3 * s[2:3, :])


def _final_body(aggf_ref, nf_ref, x0_ref, w_ref, b_ref, g_ref, be_ref, o_ref):
    aggf = aggf_ref[...]
    sinkf = jnp.logical_not(jnp.isneginf(aggf[:, 0:1]))
    aggf = jnp.where(jnp.isneginf(aggf), 0.0, aggf)
    nf = nf_ref[...]
    out = jnp.dot(aggf, w_ref[...], preferred_element_type=jnp.float32) + b_ref[...]
    out = _ln(out, g_ref[...], be_ref[...])
    subf = jax.nn.relu(out + nf)
    neginf = jnp.float32(-jnp.inf)
    v2m = jnp.max(jnp.where(sinkf, subf, neginf), axis=0, keepdims=True)
    v2 = jnp.where(jnp.any(sinkf), v2m, jnp.full((1, H), 1e-4, jnp.float32))
    rem = jnp.logical_not(x0_ref[...] > 0.1)
    v3m = jnp.max(jnp.where(rem, nf, neginf), axis=0, keepdims=True)
    v3 = jnp.where(jnp.any(rem), v3m, jnp.full((1, H), 1e-4, jnp.float32))
    o_ref[0:1, :] = v2
    o_ref[1:2, :] = v3


def _tc_in(x, W, b):
    return pl.pallas_call(
        _in_body,
        out_shape=jax.ShapeDtypeStruct((N, H), jnp.float32),
    )(x, W, b.reshape(1, H))


def _tc_layer(agg, h, W, b, g, be):
    return pl.pallas_call(
        _layer_body,
        out_shape=jax.ShapeDtypeStruct((N, H), jnp.float32),
    )(agg, h, W, b.reshape(1, H), g.reshape(1, H), be.reshape(1, H))


def _tc_layer3(agg, h1, h2, W, b, g, be, gates):
    return pl.pallas_call(
        _layer3_body,
        out_shape=jax.ShapeDtypeStruct((N, H), jnp.float32),
    )(agg, h1, h2, W, b.reshape(1, H), g.reshape(1, H), be.reshape(1, H),
      gates)


def _tc_final(aggf, nf, x0, W, b, g, be):
    return pl.pallas_call(
        _final_body,
        out_shape=jax.ShapeDtypeStruct((2, H), jnp.float32),
    )(aggf, nf, x0, W, b.reshape(1, H), g.reshape(1, H), be.reshape(1, H))


def kernel(x, edge_index, edge_attr, W_in, b_in, W1, b1, g1, be1, W2, b2, g2,
           be2, W3, b3, g3, be3, Wf, bf, gf, bef, gates):
    src = edge_index[0]
    dst = edge_index[1]
    lst, plst, cnt, pcnt = _sc_prepare(src, dst, edge_attr)
    h0 = _tc_in(x, W_in, b_in)
    agg1 = _sc_apply(h0, lst, cnt)
    h1 = _tc_layer(agg1, h0, W1, b1, g1, be1)
    agg2 = _sc_apply(h1, lst, cnt)
    h2 = _tc_layer(agg2, h1, W2, b2, g2, be2)
    agg3 = _sc_apply(h2, lst, cnt)
    nf = _tc_layer3(agg3, h1, h2, W3, b3, g3, be3, gates)
    aggf = _sc_apply(nf, plst, pcnt)
    return _tc_final(aggf, nf, x[:, 0:1], Wf, bf, gf, bef)
